# baseline (speedup 1.0000x reference)
import jax
import jax.numpy as jnp
from jax.experimental import pallas as pl

B = 8192
H = 2048
DIN = 3 * H
DOUT = 2048
BM = 256
BN = 1024
MB = B // BM
NB = DOUT // BN
BF = jnp.bfloat16


def _leaky(x):
    return jnp.where(x >= 0, x, 0.01 * x)


def _k_layer1(x1, x2, x3, w, b, out):
    xa = x1[...].astype(BF)
    xb = x2[...].astype(BF)
    xc = x3[...].astype(BF)
    W = w[0]
    acc = jnp.dot(xa, W[0:H], preferred_element_type=jnp.float32)
    acc += jnp.dot(xb, W[H:2 * H], preferred_element_type=jnp.float32)
    acc += jnp.dot(xc, W[2 * H:3 * H], preferred_element_type=jnp.float32)
    acc += b[0]
    out[...] = _leaky(acc).astype(BF)


def _k_layers234(h1, w2, b2, w3, b3, w4, b4, out):
    x = h1[...]
    h2 = _leaky(jnp.dot(x, w2[0], preferred_element_type=jnp.float32) + b2[0]).astype(BF)
    h3 = _leaky(jnp.dot(h2, w3[0], preferred_element_type=jnp.float32) + b3[0]).astype(BF)
    out[...] = jnp.dot(h3, w4[0], preferred_element_type=jnp.float32) + b4[0]


def kernel(l_ty, l_ey, l_y, t,
           W1_l1, b1_l1, W2_l1, b2_l1, W3_l1, b3_l1, W4_l1, b4_l1,
           W1_l2, b1_l2, W2_l2, b2_l2, W3_l2, b3_l2, W4_l2, b4_l2):
    W1s = jnp.stack([W1_l1.T, W1_l2.T]).astype(BF)
    W2s = jnp.stack([W2_l1.T, W2_l2.T]).astype(BF)
    W3s = jnp.stack([W3_l1.T, W3_l2.T]).astype(BF)
    W4s = jnp.pad(jnp.stack([W4_l1.T, W4_l2.T]),
                  ((0, 0), (0, 0), (0, 127))).astype(BF)
    b1s = jnp.stack([b1_l1, b1_l2])[:, None, :]
    b2s = jnp.stack([b2_l1, b2_l2])[:, None, :]
    b3s = jnp.stack([b3_l1, b3_l2])[:, None, :]
    b4s = jnp.pad(jnp.stack([b4_l1, b4_l2])[:, None, :],
                  ((0, 0), (0, 0), (0, 127)))

    h1 = pl.pallas_call(
        _k_layer1,
        grid=(2, NB, MB),
        in_specs=[
            pl.BlockSpec((BM, H), lambda l, n, m: (m, 0)),
            pl.BlockSpec((BM, H), lambda l, n, m: (m, 0)),
            pl.BlockSpec((BM, H), lambda l, n, m: (m, 0)),
            pl.BlockSpec((1, DIN, BN), lambda l, n, m: (l, 0, n)),
            pl.BlockSpec((1, 1, BN), lambda l, n, m: (l, 0, n)),
        ],
        out_specs=pl.BlockSpec((BM, BN), lambda l, n, m: (l * MB + m, n)),
        out_shape=jax.ShapeDtypeStruct((2 * B, DOUT), BF),
    )(l_ty, l_ey, l_y, W1s, b1s)

    out4 = pl.pallas_call(
        _k_layers234,
        grid=(2, MB),
        in_specs=[
            pl.BlockSpec((BM, DOUT), lambda l, m: (l * MB + m, 0)),
            pl.BlockSpec((1, DOUT, DOUT), lambda l, m: (l, 0, 0)),
            pl.BlockSpec((1, 1, DOUT), lambda l, m: (l, 0, 0)),
            pl.BlockSpec((1, DOUT, DOUT), lambda l, m: (l, 0, 0)),
            pl.BlockSpec((1, 1, DOUT), lambda l, m: (l, 0, 0)),
            pl.BlockSpec((1, DOUT, 128), lambda l, m: (l, 0, 0)),
            pl.BlockSpec((1, 1, 128), lambda l, m: (l, 0, 0)),
        ],
        out_specs=pl.BlockSpec((BM, 128), lambda l, m: (l * MB + m, 0)),
        out_shape=jax.ShapeDtypeStruct((2 * B, 128), jnp.float32),
    )(h1, W2s, b2s, W3s, b3s, W4s, b4s)

    base = jax.random.uniform(jax.random.key(1), (B,), dtype=jnp.float32)
    tf = t[:, 0]
    v1 = out4[0:B, 0]
    v2 = out4[B:2 * B, 0]
    return jnp.where(tf == 1, v1, jnp.where(tf == 2, v2, base))
